# baseline (speedup 1.0000x reference)
import functools

import jax
import jax.numpy as jnp
from jax import lax
from jax.experimental import pallas as pl
from jax.experimental.pallas import tpu as pltpu
from jax.experimental.pallas import tpu_sc as plsc

N = 10000
E = 320000
D = 128

NC = 2
NS = 16
NW = NC * NS
EPW = E // NW
CH = 80
NCH = EPW // CH

SL = 5
ES = E // SL
ESW = ES // NW
NCHS = ESW // CH
NPAD = 10240
RPT = NPAD // NS

BE = 2560
GE = E // BE

@functools.cache
def _mesh():
    return plsc.VectorSubcoreMesh(
        core_axis_name="c", subcore_axis_name="s", num_cores=NC, num_subcores=NS
    )


def _mul_rows(a_buf, b_buf, n=None):
    def row(i, carry):
        for j in range(D // 16):
            sl = pl.ds(j * 16, 16)
            a_buf[i, sl] = a_buf[i, sl] * b_buf[i, sl]
        return carry

    lax.fori_loop(0, CH if n is None else n, row, 0)


RING = 4


def _sc_gather_body(q_hbm, k_hbm, dst_hbm, src_hbm, oqk_hbm,
                    dsti, srci, qbufs, kbufs, gsems, wsems):
    wid = lax.axis_index("s") * NC + lax.axis_index("c")
    base = wid * ESW

    pltpu.sync_copy(dst_hbm.at[pl.ds(base, ESW)], dsti)
    pltpu.sync_copy(src_hbm.at[pl.ds(base, ESW)], srci)

    def issue(j, b):
        loc = j * CH
        pltpu.async_copy(q_hbm.at[dsti.at[pl.ds(loc, CH)]], qbufs[b], gsems[b])
        pltpu.async_copy(k_hbm.at[srci.at[pl.ds(loc, CH)]], kbufs[b], gsems[b])

    def wait_gather(j, b):
        loc = j * CH
        pltpu.make_async_copy(q_hbm.at[dsti.at[pl.ds(loc, CH)]], qbufs[b], gsems[b]).wait()
        pltpu.make_async_copy(k_hbm.at[srci.at[pl.ds(loc, CH)]], kbufs[b], gsems[b]).wait()

    def wdesc(j, b):
        return pltpu.make_async_copy(
            qbufs[b], oqk_hbm.at[pl.ds(base + j * CH, CH)], wsems[b])

    def body(i, carry):
        for b in range(RING):
            j = i * RING + b

            @pl.when(i > 0)
            def _():
                wdesc(j - RING, b).wait()

            issue(j, b)
        for b in range(RING):
            j = i * RING + b
            wait_gather(j, b)
            _mul_rows(qbufs[b], kbufs[b])
            pltpu.async_copy(qbufs[b], oqk_hbm.at[pl.ds(base + j * CH, CH)],
                             wsems[b])
        return carry

    nloop = NCHS // RING
    lax.fori_loop(0, nloop, body, 0)
    for b in range(RING):
        wdesc((nloop - 1) * RING + b, b).wait()
    for j in range(nloop * RING, NCHS):
        b = j - nloop * RING
        issue(j, b)
        wait_gather(j, b)
        _mul_rows(qbufs[b], kbufs[b])
        pltpu.sync_copy(qbufs[b], oqk_hbm.at[pl.ds(base + j * CH, CH)])


@functools.cache
def _sc_gather_kernel():
    return pl.kernel(
        _sc_gather_body,
        out_type=jax.ShapeDtypeStruct((ES, D), jnp.float32),
        mesh=_mesh(),
        scratch_types=[
            pltpu.VMEM((ESW,), jnp.int32),
            pltpu.VMEM((ESW,), jnp.int32),
            [pltpu.VMEM((CH, D), jnp.float32) for _ in range(RING)],
            [pltpu.VMEM((CH, D), jnp.float32) for _ in range(RING)],
            [pltpu.SemaphoreType.DMA for _ in range(RING)],
            [pltpu.SemaphoreType.DMA for _ in range(RING)],
        ],
    )


def _sc_gather(q_tab, k_tab, dst, src):
    return _sc_gather_kernel()(q_tab, k_tab, dst, src)


NFC = RPT // CH


def _zero_table(zero_hbm, buf, table, row0, rows=CH):
    pltpu.sync_copy(zero_hbm, buf)
    for i in range(RPT // rows):
        pltpu.sync_copy(buf, table.at[pl.ds(row0 + i * rows, rows)])


def _flush_table(table, buf, out_hbm, row0, out0, rows=CH):
    for i in range(RPT // rows):
        pltpu.sync_copy(table.at[pl.ds(row0 + i * rows, rows)], buf)
        pltpu.sync_copy(buf, out_hbm.at[pl.ds(out0 + i * rows, rows)])


ZRING = 4


def _z_slice_ring(data_hbm, dst_hbm, base, dstis, dbufs, isems, dsems, ssems,
                  table):
    def loads(j, b):
        off = base + j * CH
        pltpu.async_copy(dst_hbm.at[pl.ds(off, CH)], dstis[b], isems[b])
        pltpu.async_copy(data_hbm.at[pl.ds(off, CH)], dbufs[b], dsems[b])

    def wait_loads(j, b):
        off = base + j * CH
        pltpu.make_async_copy(dst_hbm.at[pl.ds(off, CH)], dstis[b], isems[b]).wait()
        pltpu.make_async_copy(data_hbm.at[pl.ds(off, CH)], dbufs[b], dsems[b]).wait()

    def sdesc(b):
        return pltpu.make_async_copy(dbufs[b], table.at[dstis[b]], ssems[b])

    def body(i, carry):
        for b in range(ZRING):
            j = i * ZRING + b

            @pl.when(i > 0)
            def _():
                sdesc(b).wait()

            loads(j, b)
        for b in range(ZRING):
            j = i * ZRING + b
            wait_loads(j, b)
            pltpu.async_copy(dbufs[b], table.at[dstis[b]], ssems[b], add=True)
        return carry

    nloop = NCHS // ZRING
    lax.fori_loop(0, nloop, body, 0)
    for b in range(ZRING):
        sdesc(b).wait()
    for j in range(nloop * ZRING, NCHS):
        b = j - nloop * ZRING
        loads(j, b)
        wait_loads(j, b)
        pltpu.sync_copy(dbufs[b], table.at[dstis[b]], add=True)


def _make_scatter_body(nsl):
    def body(*refs):
        data_refs = refs[0:nsl]
        dst_refs = refs[nsl:2 * nsl]
        zero_hbm = refs[2 * nsl]
        out_hbm = refs[2 * nsl + 1]
        (dstis, dbufs, isems, dsems, ssems, table) = refs[2 * nsl + 2:]

        c = lax.axis_index("c")
        s = lax.axis_index("s")
        base = (s * NC + c) * ESW
        row0 = s * RPT

        _zero_table(zero_hbm, dbufs[0], table, row0)
        plsc.subcore_barrier()
        for sl in range(nsl):
            _z_slice_ring(data_refs[sl], dst_refs[sl], base,
                          dstis, dbufs, isems, dsems, ssems, table)
        plsc.subcore_barrier()
        _flush_table(table, dbufs[0], out_hbm, row0, c * NPAD + row0)

    return body


@functools.cache
def _sc_scatter_kernel(nsl):
    return pl.kernel(
        _make_scatter_body(nsl),
        out_type=jax.ShapeDtypeStruct((NC * NPAD, D), jnp.float32),
        mesh=_mesh(),
        scratch_types=[
            [pltpu.VMEM((CH,), jnp.int32) for _ in range(ZRING)],
            [pltpu.VMEM((CH, D), jnp.float32) for _ in range(ZRING)],
            [pltpu.SemaphoreType.DMA for _ in range(ZRING)],
            [pltpu.SemaphoreType.DMA for _ in range(ZRING)],
            [pltpu.SemaphoreType.DMA for _ in range(ZRING)],
            pltpu.VMEM_SHARED((NPAD, D), jnp.float32),
        ],
    )


def _sc_scatter(data_slices, dst_slices, zeros):
    nsl = len(data_slices)
    return _sc_scatter_kernel(nsl)(*data_slices, *dst_slices, zeros)


VRING = 4


CHV = 40
NCHSV = ESW // CHV


def _v_slice_ring(ef_hbm, vtab_hbm, dst_hbm, src_hbm, base,
                  dstis, srcis, dbufs, vbufs,
                  isems, jsems, dsems, gsems, ssems, table):
    def loads(j, b):
        off = base + j * CHV
        pltpu.async_copy(dst_hbm.at[pl.ds(off, CHV)], dstis[b], isems[b])
        pltpu.async_copy(src_hbm.at[pl.ds(off, CHV)], srcis[b], jsems[b])
        pltpu.async_copy(ef_hbm.at[pl.ds(off, CHV)], dbufs[b], dsems[b])

    def wait_src(j, b):
        off = base + j * CHV
        pltpu.make_async_copy(src_hbm.at[pl.ds(off, CHV)], srcis[b], jsems[b]).wait()

    def wait_rest(j, b):
        off = base + j * CHV
        pltpu.make_async_copy(dst_hbm.at[pl.ds(off, CHV)], dstis[b], isems[b]).wait()
        pltpu.make_async_copy(ef_hbm.at[pl.ds(off, CHV)], dbufs[b], dsems[b]).wait()
        pltpu.make_async_copy(vtab_hbm.at[srcis[b]], vbufs[b], gsems[b]).wait()

    def sdesc(b):
        return pltpu.make_async_copy(dbufs[b], table.at[dstis[b]], ssems[b])

    def body(i, carry):
        for b in range(VRING):
            j = i * VRING + b

            @pl.when(i > 0)
            def _():
                sdesc(b).wait()

            loads(j, b)
        for b in range(VRING):
            j = i * VRING + b
            wait_src(j, b)
            pltpu.async_copy(vtab_hbm.at[srcis[b]], vbufs[b], gsems[b])
        for b in range(VRING):
            j = i * VRING + b
            wait_rest(j, b)
            _mul_rows(dbufs[b], vbufs[b], CHV)
            pltpu.async_copy(dbufs[b], table.at[dstis[b]], ssems[b], add=True)
        return carry

    nloop = NCHSV // VRING
    lax.fori_loop(0, nloop, body, 0)
    for b in range(VRING):
        sdesc(b).wait()
    for j in range(nloop * VRING, NCHSV):
        b = j - nloop * VRING
        loads(j, b)
        wait_src(j, b)
        pltpu.async_copy(vtab_hbm.at[srcis[b]], vbufs[b], gsems[b])
        wait_rest(j, b)
        _mul_rows(dbufs[b], vbufs[b], CHV)
        pltpu.sync_copy(dbufs[b], table.at[dstis[b]], add=True)


def _make_scatter_mul_body(nsl):
    def body(*refs):
        ef_refs = refs[0:nsl]
        dst_refs = refs[nsl:2 * nsl]
        src_refs = refs[2 * nsl:3 * nsl]
        vtab_hbm = refs[3 * nsl]
        zero_hbm = refs[3 * nsl + 1]
        out_hbm = refs[3 * nsl + 2]
        (dstis, srcis, dbufs, vbufs, isems, jsems, dsems, gsems, ssems,
         table) = refs[3 * nsl + 3:]

        c = lax.axis_index("c")
        s = lax.axis_index("s")
        base = (s * NC + c) * ESW
        row0 = s * RPT

        _zero_table(zero_hbm, dbufs[0], table, row0, CHV)
        plsc.subcore_barrier()
        for sl in range(nsl):
            _v_slice_ring(ef_refs[sl], vtab_hbm, dst_refs[sl], src_refs[sl],
                          base, dstis, srcis, dbufs, vbufs,
                          isems, jsems, dsems, gsems, ssems, table)
        plsc.subcore_barrier()
        _flush_table(table, dbufs[0], out_hbm, row0, c * NPAD + row0, CHV)

    return body


@functools.cache
def _sc_scatter_mul_kernel(nsl):
    return pl.kernel(
        _make_scatter_mul_body(nsl),
        out_type=jax.ShapeDtypeStruct((NC * NPAD, D), jnp.float32),
        mesh=_mesh(),
        scratch_types=[
            [pltpu.VMEM((CHV,), jnp.int32) for _ in range(VRING)],
            [pltpu.VMEM((CHV,), jnp.int32) for _ in range(VRING)],
            [pltpu.VMEM((CHV, D), jnp.float32) for _ in range(VRING)],
            [pltpu.VMEM((CHV, D), jnp.float32) for _ in range(VRING)],
            [pltpu.SemaphoreType.DMA for _ in range(VRING)],
            [pltpu.SemaphoreType.DMA for _ in range(VRING)],
            [pltpu.SemaphoreType.DMA for _ in range(VRING)],
            [pltpu.SemaphoreType.DMA for _ in range(VRING)],
            [pltpu.SemaphoreType.DMA for _ in range(VRING)],
            pltpu.VMEM_SHARED((NPAD, D), jnp.float32),
        ],
    )


def _sc_scatter_mul(ef_slices, v_tab, dst_slices, src_slices, zeros):
    nsl = len(ef_slices)
    return _sc_scatter_mul_kernel(nsl)(*ef_slices, *dst_slices, *src_slices,
                                       v_tab, zeros)


def _qkv_body(nf_ref, wq_ref, wk_ref, wv_ref, q_ref, k_ref, v_ref):
    x = nf_ref[...]
    q_ref[...] = jnp.dot(x, wq_ref[...], preferred_element_type=jnp.float32)
    k_ref[...] = jnp.dot(x, wk_ref[...], preferred_element_type=jnp.float32)
    v_ref[...] = jnp.dot(x, wv_ref[...], preferred_element_type=jnp.float32)


def _qkv_call(nf, wq, wk, wv):
    sd = jax.ShapeDtypeStruct((N, D), jnp.float32)
    return pl.pallas_call(
        _qkv_body,
        out_shape=(sd, sd, sd),
    )(nf, wq, wk, wv)


def _bdot(a, b):
    return jnp.dot(a.astype(jnp.bfloat16), b.astype(jnp.bfloat16),
                   preferred_element_type=jnp.float32)


def _edge1_compute(qk_ref, x_ref, we_ref, ow_ref, ob_ref,
                   ef_ref, t_ref, acc_ref):
    i = pl.program_id(0)
    x = x_ref[...]
    ep = _bdot(x, we_ref[...])
    att = jnp.clip(qk_ref[...] * 0.25, -5.0, 5.0)
    ef = jnp.clip(jnp.exp(att * ep), -5.0, 5.0)
    t = x + _bdot(ef, ow_ref[...]) + ob_ref[...]
    ef_ref[...] = ef
    t_ref[...] = t.astype(jnp.bfloat16)

    @pl.when(i == 0)
    def _():
        acc_ref[...] = jnp.zeros_like(acc_ref)

    acc_ref[0:1, :] += jnp.sum(t, axis=0, keepdims=True)
    acc_ref[1:2, :] += jnp.sum(t * t, axis=0, keepdims=True)


def _edge1_body0(qk_ref, x_ref, we_ref, ow_ref, ob_ref, ef_ref, t_ref, acc_ref):
    _edge1_compute(qk_ref, x_ref, we_ref, ow_ref, ob_ref, ef_ref, t_ref, acc_ref)


def _edge1_bodyN(qk_ref, x_ref, t_al, we_ref, ow_ref, ob_ref,
                 ef_ref, t_ref, acc_ref):
    _edge1_compute(qk_ref, x_ref, we_ref, ow_ref, ob_ref, ef_ref, t_ref, acc_ref)


GS = ES // BE


def _edge1_call(s, qk_s, edge_feat, t_prev, we, ow, ob):
    soff = s * GS
    loc = pl.BlockSpec((BE, D), lambda i: (i, 0))
    glob = pl.BlockSpec((BE, D), lambda i: (soff + i, 0))
    full = lambda r, c: pl.BlockSpec((r, c), lambda i: (0, 0))
    anyspec = pl.BlockSpec(memory_space=pl.ANY)
    out_shape = [
        jax.ShapeDtypeStruct((ES, D), jnp.float32),
        jax.ShapeDtypeStruct((E, D), jnp.bfloat16),
        jax.ShapeDtypeStruct((8, D), jnp.float32),
    ]
    if s == 0:
        return pl.pallas_call(
            _edge1_body0,
            grid=(GS,),
            in_specs=[loc, glob, full(D, D), full(D, D), full(1, D)],
            out_specs=[loc, glob, full(8, D)],
            out_shape=out_shape,
        )(qk_s, edge_feat, we, ow, ob)
    return pl.pallas_call(
        _edge1_bodyN,
        grid=(GS,),
        in_specs=[loc, glob, anyspec, full(D, D), full(D, D), full(1, D)],
        out_specs=[loc, glob, full(8, D)],
        out_shape=out_shape,
        input_output_aliases={2: 1},
    )(qk_s, edge_feat, t_prev, we, ow, ob)


def _edge2_body(t_ref, acc_ref, w1_ref, b1_ref, w2_ref, b2_ref, g_ref, bb_ref,
                u_ref, acc2_ref):
    i = pl.program_id(0)
    a = acc_ref[...]
    mu = jnp.sum(a[:, 0, :], axis=0, keepdims=True) * (1.0 / E)
    var = jnp.sum(a[:, 1, :], axis=0, keepdims=True) * (1.0 / E) - mu * mu
    inv = g_ref[...] * jax.lax.rsqrt(var + 1e-5)
    e1 = (t_ref[...].astype(jnp.float32) - mu) * inv + bb_ref[...]
    hid = jnp.maximum(_bdot(e1, w1_ref[...]) + b1_ref[...], 0.0)
    u = e1 + _bdot(hid, w2_ref[...]) + b2_ref[...]
    u_ref[...] = u.astype(jnp.bfloat16)

    @pl.when(i == 0)
    def _():
        acc2_ref[...] = jnp.zeros_like(acc2_ref)

    acc2_ref[0:1, :] += jnp.sum(u, axis=0, keepdims=True)
    acc2_ref[1:2, :] += jnp.sum(u * u, axis=0, keepdims=True)


def _edge2_call(t, acc, w1, b1, w2, b2, g, bb):
    blk = lambda w: pl.BlockSpec((BE, w), lambda i: (i, 0))
    full = lambda r, c: pl.BlockSpec((r, c), lambda i: (0, 0))
    acc_spec = pl.BlockSpec((SL, 8, D), lambda i: (0, 0, 0))
    return pl.pallas_call(
        _edge2_body,
        grid=(GE,),
        in_specs=[blk(D), acc_spec, full(D, 2 * D), full(1, 2 * D),
                  full(2 * D, D), full(1, D), full(1, D), full(1, D)],
        out_specs=[blk(D), full(8, D)],
        out_shape=[
            jax.ShapeDtypeStruct((E, D), jnp.bfloat16),
            jax.ShapeDtypeStruct((8, D), jnp.float32),
        ],
    )(t, acc, w1, b1, w2, b2, g, bb)


def _edge3_body(u_ref, acc_ref, g_ref, bb_ref, e_ref):
    mu = acc_ref[0:1, :] * (1.0 / E)
    var = acc_ref[1:2, :] * (1.0 / E) - mu * mu
    inv = g_ref[...] * jax.lax.rsqrt(var + 1e-5)
    e_ref[...] = (u_ref[...].astype(jnp.float32) - mu) * inv + bb_ref[...]


def _edge3_call(u, acc, g, bb):
    blk = lambda w: pl.BlockSpec((BE, w), lambda i: (i, 0))
    full = lambda r, c: pl.BlockSpec((r, c), lambda i: (0, 0))
    return pl.pallas_call(
        _edge3_body,
        grid=(GE,),
        in_specs=[blk(D), full(8, D), full(1, D), full(1, D)],
        out_specs=blk(D),
        out_shape=jax.ShapeDtypeStruct((E, D), jnp.float32),
    )(u, acc, g, bb)


def _node_body(zpa_ref, vpa_ref, nf_ref, ow_ref, ob_ref,
               w1_ref, b1_ref, w2_ref, b2_ref,
               g1_ref, bb1_ref, g2_ref, bb2_ref, h_ref):
    z = zpa_ref[:N, :] + zpa_ref[NPAD:NPAD + N, :]
    v = vpa_ref[:N, :] + vpa_ref[NPAD:NPAD + N, :]
    h_attn = v / z + 1e-6
    h = nf_ref[...] + jnp.dot(h_attn, ow_ref[...],
                              preferred_element_type=jnp.float32) + ob_ref[...]
    mu = jnp.mean(h, axis=0, keepdims=True)
    var = jnp.mean((h - mu) * (h - mu), axis=0, keepdims=True)
    h = g1_ref[...] * (h - mu) * jax.lax.rsqrt(var + 1e-5) + bb1_ref[...]
    hid = jnp.maximum(
        jnp.dot(h, w1_ref[...], preferred_element_type=jnp.float32) + b1_ref[...],
        0.0,
    )
    h2 = h + jnp.dot(hid, w2_ref[...], preferred_element_type=jnp.float32) + b2_ref[...]
    mu2 = jnp.mean(h2, axis=0, keepdims=True)
    var2 = jnp.mean((h2 - mu2) * (h2 - mu2), axis=0, keepdims=True)
    h_ref[...] = g2_ref[...] * (h2 - mu2) * jax.lax.rsqrt(var2 + 1e-5) + bb2_ref[...]


def _node_call(zpa, vpa, nf, ow, ob, w1, b1, w2, b2, g1, bb1, g2, bb2):
    return pl.pallas_call(
        _node_body,
        out_shape=jax.ShapeDtypeStruct((N, D), jnp.float32),
    )(zpa, vpa, nf, ow, ob, w1, b1, w2, b2, g1, bb1, g2, bb2)


def kernel(node_feat, edge_feat, edge_index, W_Q, W_K, W_V, W_E,
           O_h_W, O_h_b, O_e_W, O_e_b,
           F_h_W1, F_h_b1, F_h_W2, F_h_b2,
           F_e_W1, F_e_b1, F_e_W2, F_e_b2,
           bn1_h_g, bn1_h_b, bn1_e_g, bn1_e_b,
           bn2_h_g, bn2_h_b, bn2_e_g, bn2_e_b):
    src = edge_index[0].astype(jnp.int32)
    dst = edge_index[1].astype(jnp.int32)
    r = lambda x: x.reshape(1, -1)

    q_tab, k_tab, v_tab = _qkv_call(node_feat, W_Q, W_K, W_V)

    dst_s = [dst[s * ES:(s + 1) * ES] for s in range(SL)]
    src_s = [src[s * ES:(s + 1) * ES] for s in range(SL)]
    qk_s = [_sc_gather(q_tab, k_tab, dst_s[s], src_s[s]) for s in range(SL)]

    t = None
    ef_s = []
    accs = []
    for s in range(SL):
        ef_i, t, acc_i = _edge1_call(s, qk_s[s], edge_feat, t,
                                     W_E, O_e_W, r(O_e_b))
        ef_s.append(ef_i)
        accs.append(acc_i)
    acc1 = jnp.stack(accs)

    zeros = jnp.zeros((CH, D), jnp.float32)
    zeros_v = jnp.zeros((CHV, D), jnp.float32)
    zpa = _sc_scatter(tuple(ef_s), tuple(dst_s), zeros)
    vpa = _sc_scatter_mul(tuple(ef_s), v_tab, tuple(dst_s),
                          tuple(src_s), zeros_v)

    u, acc2 = _edge2_call(t, acc1, F_e_W1, r(F_e_b1), F_e_W2, r(F_e_b2),
                          r(bn1_e_g), r(bn1_e_b))
    e_out = _edge3_call(u, acc2, r(bn2_e_g), r(bn2_e_b))

    h_out = _node_call(zpa, vpa, node_feat, O_h_W, r(O_h_b),
                       F_h_W1, r(F_h_b1), F_h_W2, r(F_h_b2),
                       r(bn1_h_g), r(bn1_h_b), r(bn2_h_g), r(bn2_h_b))
    return (h_out, e_out)


# speedup vs baseline: 1.0337x; 1.0337x over previous
import functools

import jax
import jax.numpy as jnp
from jax import lax
from jax.experimental import pallas as pl
from jax.experimental.pallas import tpu as pltpu
from jax.experimental.pallas import tpu_sc as plsc

N = 10000
E = 320000
D = 128

NC = 2
NS = 16
NW = NC * NS
EPW = E // NW
CH = 80
NCH = EPW // CH

SL = 5
ES = E // SL
ESW = ES // NW
NCHS = ESW // CH
NPAD = 10240
RPT = NPAD // NS

BE = 2560
GE = E // BE

@functools.cache
def _mesh():
    return plsc.VectorSubcoreMesh(
        core_axis_name="c", subcore_axis_name="s", num_cores=NC, num_subcores=NS
    )


def _mul_rows(a_buf, b_buf):
    def row(i, carry):
        for j in range(D // 16):
            sl = pl.ds(j * 16, 16)
            a_buf[i, sl] = a_buf[i, sl] * b_buf[i, sl]
        return carry

    lax.fori_loop(0, CH, row, 0)


RING = 4


def _sc_gather_body(q_hbm, k_hbm, dst_hbm, src_hbm, oqk_hbm,
                    dsti, srci, qbufs, kbufs, gsems, wsems):
    wid = lax.axis_index("s") * NC + lax.axis_index("c")
    base = wid * ESW

    pltpu.sync_copy(dst_hbm.at[pl.ds(base, ESW)], dsti)
    pltpu.sync_copy(src_hbm.at[pl.ds(base, ESW)], srci)

    def issue(j, b):
        loc = j * CH
        pltpu.async_copy(q_hbm.at[dsti.at[pl.ds(loc, CH)]], qbufs[b], gsems[b])
        pltpu.async_copy(k_hbm.at[srci.at[pl.ds(loc, CH)]], kbufs[b], gsems[b])

    def wait_gather(j, b):
        loc = j * CH
        pltpu.make_async_copy(q_hbm.at[dsti.at[pl.ds(loc, CH)]], qbufs[b], gsems[b]).wait()
        pltpu.make_async_copy(k_hbm.at[srci.at[pl.ds(loc, CH)]], kbufs[b], gsems[b]).wait()

    def wdesc(j, b):
        return pltpu.make_async_copy(
            qbufs[b], oqk_hbm.at[pl.ds(base + j * CH, CH)], wsems[b])

    def body(i, carry):
        for b in range(RING):
            j = i * RING + b

            @pl.when(i > 0)
            def _():
                wdesc(j - RING, b).wait()

            issue(j, b)
        for b in range(RING):
            j = i * RING + b
            wait_gather(j, b)
            _mul_rows(qbufs[b], kbufs[b])
            pltpu.async_copy(qbufs[b], oqk_hbm.at[pl.ds(base + j * CH, CH)],
                             wsems[b])
        return carry

    nloop = NCHS // RING
    lax.fori_loop(0, nloop, body, 0)
    for b in range(RING):
        wdesc((nloop - 1) * RING + b, b).wait()
    for j in range(nloop * RING, NCHS):
        b = j - nloop * RING
        issue(j, b)
        wait_gather(j, b)
        _mul_rows(qbufs[b], kbufs[b])
        pltpu.sync_copy(qbufs[b], oqk_hbm.at[pl.ds(base + j * CH, CH)])


@functools.cache
def _sc_gather_kernel():
    return pl.kernel(
        _sc_gather_body,
        out_type=jax.ShapeDtypeStruct((ES, D), jnp.float32),
        mesh=_mesh(),
        scratch_types=[
            pltpu.VMEM((ESW,), jnp.int32),
            pltpu.VMEM((ESW,), jnp.int32),
            [pltpu.VMEM((CH, D), jnp.float32) for _ in range(RING)],
            [pltpu.VMEM((CH, D), jnp.float32) for _ in range(RING)],
            [pltpu.SemaphoreType.DMA for _ in range(RING)],
            [pltpu.SemaphoreType.DMA for _ in range(RING)],
        ],
    )


def _sc_gather(q_tab, k_tab, dst, src):
    return _sc_gather_kernel()(q_tab, k_tab, dst, src)


NFC = RPT // CH


def _zero_table(zero_hbm, buf, table, row0):
    pltpu.sync_copy(zero_hbm, buf)
    for i in range(NFC):
        pltpu.sync_copy(buf, table.at[pl.ds(row0 + i * CH, CH)])


def _flush_table(table, buf, out_hbm, row0, out0):
    for i in range(NFC):
        pltpu.sync_copy(table.at[pl.ds(row0 + i * CH, CH)], buf)
        pltpu.sync_copy(buf, out_hbm.at[pl.ds(out0 + i * CH, CH)])


ZRING = 4


def _z_slice_ring(data_hbm, dst_hbm, base, dstis, dbufs, isems, dsems, ssems,
                  table):
    def loads(j, b):
        off = base + j * CH
        pltpu.async_copy(dst_hbm.at[pl.ds(off, CH)], dstis[b], isems[b])
        pltpu.async_copy(data_hbm.at[pl.ds(off, CH)], dbufs[b], dsems[b])

    def wait_loads(j, b):
        off = base + j * CH
        pltpu.make_async_copy(dst_hbm.at[pl.ds(off, CH)], dstis[b], isems[b]).wait()
        pltpu.make_async_copy(data_hbm.at[pl.ds(off, CH)], dbufs[b], dsems[b]).wait()

    def sdesc(b):
        return pltpu.make_async_copy(dbufs[b], table.at[dstis[b]], ssems[b])

    def body(i, carry):
        for b in range(ZRING):
            j = i * ZRING + b

            @pl.when(i > 0)
            def _():
                sdesc(b).wait()

            loads(j, b)
        for b in range(ZRING):
            j = i * ZRING + b
            wait_loads(j, b)
            pltpu.async_copy(dbufs[b], table.at[dstis[b]], ssems[b], add=True)
        return carry

    nloop = NCHS // ZRING
    lax.fori_loop(0, nloop, body, 0)
    for b in range(ZRING):
        sdesc(b).wait()
    for j in range(nloop * ZRING, NCHS):
        b = j - nloop * ZRING
        loads(j, b)
        wait_loads(j, b)
        pltpu.sync_copy(dbufs[b], table.at[dstis[b]], add=True)


def _make_scatter_body(nsl):
    def body(*refs):
        data_refs = refs[0:nsl]
        dst_refs = refs[nsl:2 * nsl]
        zero_hbm = refs[2 * nsl]
        out_hbm = refs[2 * nsl + 1]
        (dstis, dbufs, isems, dsems, ssems, table) = refs[2 * nsl + 2:]

        c = lax.axis_index("c")
        s = lax.axis_index("s")
        base = (s * NC + c) * ESW
        row0 = s * RPT

        _zero_table(zero_hbm, dbufs[0], table, row0)
        plsc.subcore_barrier()
        for sl in range(nsl):
            _z_slice_ring(data_refs[sl], dst_refs[sl], base,
                          dstis, dbufs, isems, dsems, ssems, table)
        plsc.subcore_barrier()
        _flush_table(table, dbufs[0], out_hbm, row0, c * NPAD + row0)

    return body


@functools.cache
def _sc_scatter_kernel(nsl):
    return pl.kernel(
        _make_scatter_body(nsl),
        out_type=jax.ShapeDtypeStruct((NC * NPAD, D), jnp.float32),
        mesh=_mesh(),
        scratch_types=[
            [pltpu.VMEM((CH,), jnp.int32) for _ in range(ZRING)],
            [pltpu.VMEM((CH, D), jnp.float32) for _ in range(ZRING)],
            [pltpu.SemaphoreType.DMA for _ in range(ZRING)],
            [pltpu.SemaphoreType.DMA for _ in range(ZRING)],
            [pltpu.SemaphoreType.DMA for _ in range(ZRING)],
            pltpu.VMEM_SHARED((NPAD, D), jnp.float32),
        ],
    )


def _sc_scatter(data_slices, dst_slices, zeros):
    nsl = len(data_slices)
    return _sc_scatter_kernel(nsl)(*data_slices, *dst_slices, zeros)


VRING = 2


def _v_slice_ring(ef_hbm, vtab_hbm, dst_hbm, src_hbm, base,
                  dstis, srcis, dbufs, vbufs,
                  isems, jsems, dsems, gsems, ssems, table):
    def loads(j, b):
        off = base + j * CH
        pltpu.async_copy(dst_hbm.at[pl.ds(off, CH)], dstis[b], isems[b])
        pltpu.async_copy(src_hbm.at[pl.ds(off, CH)], srcis[b], jsems[b])
        pltpu.async_copy(ef_hbm.at[pl.ds(off, CH)], dbufs[b], dsems[b])

    def wait_src(j, b):
        off = base + j * CH
        pltpu.make_async_copy(src_hbm.at[pl.ds(off, CH)], srcis[b], jsems[b]).wait()

    def wait_rest(j, b):
        off = base + j * CH
        pltpu.make_async_copy(dst_hbm.at[pl.ds(off, CH)], dstis[b], isems[b]).wait()
        pltpu.make_async_copy(ef_hbm.at[pl.ds(off, CH)], dbufs[b], dsems[b]).wait()
        pltpu.make_async_copy(vtab_hbm.at[srcis[b]], vbufs[b], gsems[b]).wait()

    def sdesc(b):
        return pltpu.make_async_copy(dbufs[b], table.at[dstis[b]], ssems[b])

    def body(i, carry):
        for b in range(VRING):
            j = i * VRING + b

            @pl.when(i > 0)
            def _():
                sdesc(b).wait()

            loads(j, b)
        for b in range(VRING):
            j = i * VRING + b
            wait_src(j, b)
            pltpu.async_copy(vtab_hbm.at[srcis[b]], vbufs[b], gsems[b])
        for b in range(VRING):
            j = i * VRING + b
            wait_rest(j, b)
            _mul_rows(dbufs[b], vbufs[b])
            pltpu.async_copy(dbufs[b], table.at[dstis[b]], ssems[b], add=True)
        return carry

    nloop = NCHS // VRING
    lax.fori_loop(0, nloop, body, 0)
    for b in range(VRING):
        sdesc(b).wait()
    for j in range(nloop * VRING, NCHS):
        b = j - nloop * VRING
        loads(j, b)
        wait_src(j, b)
        pltpu.async_copy(vtab_hbm.at[srcis[b]], vbufs[b], gsems[b])
        wait_rest(j, b)
        _mul_rows(dbufs[b], vbufs[b])
        pltpu.sync_copy(dbufs[b], table.at[dstis[b]], add=True)


def _make_scatter_mul_body(nsl):
    def body(*refs):
        ef_refs = refs[0:nsl]
        dst_refs = refs[nsl:2 * nsl]
        src_refs = refs[2 * nsl:3 * nsl]
        vtab_hbm = refs[3 * nsl]
        zero_hbm = refs[3 * nsl + 1]
        out_hbm = refs[3 * nsl + 2]
        (dstis, srcis, dbufs, vbufs, isems, jsems, dsems, gsems, ssems,
         table) = refs[3 * nsl + 3:]

        c = lax.axis_index("c")
        s = lax.axis_index("s")
        base = (s * NC + c) * ESW
        row0 = s * RPT

        _zero_table(zero_hbm, dbufs[0], table, row0)
        plsc.subcore_barrier()
        for sl in range(nsl):
            _v_slice_ring(ef_refs[sl], vtab_hbm, dst_refs[sl], src_refs[sl],
                          base, dstis, srcis, dbufs, vbufs,
                          isems, jsems, dsems, gsems, ssems, table)
        plsc.subcore_barrier()
        _flush_table(table, dbufs[0], out_hbm, row0, c * NPAD + row0)

    return body


@functools.cache
def _sc_scatter_mul_kernel(nsl):
    return pl.kernel(
        _make_scatter_mul_body(nsl),
        out_type=jax.ShapeDtypeStruct((NC * NPAD, D), jnp.float32),
        mesh=_mesh(),
        scratch_types=[
            [pltpu.VMEM((CH,), jnp.int32) for _ in range(VRING)],
            [pltpu.VMEM((CH,), jnp.int32) for _ in range(VRING)],
            [pltpu.VMEM((CH, D), jnp.float32) for _ in range(VRING)],
            [pltpu.VMEM((CH, D), jnp.float32) for _ in range(VRING)],
            [pltpu.SemaphoreType.DMA for _ in range(VRING)],
            [pltpu.SemaphoreType.DMA for _ in range(VRING)],
            [pltpu.SemaphoreType.DMA for _ in range(VRING)],
            [pltpu.SemaphoreType.DMA for _ in range(VRING)],
            [pltpu.SemaphoreType.DMA for _ in range(VRING)],
            pltpu.VMEM_SHARED((NPAD, D), jnp.float32),
        ],
    )


def _sc_scatter_mul(ef_slices, v_tab, dst_slices, src_slices, zeros):
    nsl = len(ef_slices)
    return _sc_scatter_mul_kernel(nsl)(*ef_slices, *dst_slices, *src_slices,
                                       v_tab, zeros)


def _qkv_body(nf_ref, wq_ref, wk_ref, wv_ref, q_ref, k_ref, v_ref):
    x = nf_ref[...]
    q_ref[...] = jnp.dot(x, wq_ref[...], preferred_element_type=jnp.float32)
    k_ref[...] = jnp.dot(x, wk_ref[...], preferred_element_type=jnp.float32)
    v_ref[...] = jnp.dot(x, wv_ref[...], preferred_element_type=jnp.float32)


def _qkv_call(nf, wq, wk, wv):
    sd = jax.ShapeDtypeStruct((N, D), jnp.float32)
    return pl.pallas_call(
        _qkv_body,
        out_shape=(sd, sd, sd),
    )(nf, wq, wk, wv)


def _bdot(a, b):
    return jnp.dot(a.astype(jnp.bfloat16), b.astype(jnp.bfloat16),
                   preferred_element_type=jnp.float32)


def _edge1_compute(qk_ref, x_ref, we_ref, ow_ref, ob_ref,
                   ef_ref, t_ref, acc_ref):
    i = pl.program_id(0)
    x = x_ref[...]
    ep = _bdot(x, we_ref[...])
    att = jnp.clip(qk_ref[...] * 0.25, -5.0, 5.0)
    ef = jnp.clip(jnp.exp(att * ep), -5.0, 5.0)
    t = x + _bdot(ef, ow_ref[...]) + ob_ref[...]
    ef_ref[...] = ef
    t_ref[...] = t.astype(jnp.bfloat16)

    @pl.when(i == 0)
    def _():
        acc_ref[...] = jnp.zeros_like(acc_ref)

    acc_ref[0:1, :] += jnp.sum(t, axis=0, keepdims=True)
    acc_ref[1:2, :] += jnp.sum(t * t, axis=0, keepdims=True)


def _edge1_body0(qk_ref, x_ref, we_ref, ow_ref, ob_ref, ef_ref, t_ref, acc_ref):
    _edge1_compute(qk_ref, x_ref, we_ref, ow_ref, ob_ref, ef_ref, t_ref, acc_ref)


def _edge1_bodyN(qk_ref, x_ref, t_al, we_ref, ow_ref, ob_ref,
                 ef_ref, t_ref, acc_ref):
    _edge1_compute(qk_ref, x_ref, we_ref, ow_ref, ob_ref, ef_ref, t_ref, acc_ref)


GS = ES // BE


def _edge1_call(s, qk_s, edge_feat, t_prev, we, ow, ob):
    soff = s * GS
    loc = pl.BlockSpec((BE, D), lambda i: (i, 0))
    glob = pl.BlockSpec((BE, D), lambda i: (soff + i, 0))
    full = lambda r, c: pl.BlockSpec((r, c), lambda i: (0, 0))
    anyspec = pl.BlockSpec(memory_space=pl.ANY)
    out_shape = [
        jax.ShapeDtypeStruct((ES, D), jnp.float32),
        jax.ShapeDtypeStruct((E, D), jnp.bfloat16),
        jax.ShapeDtypeStruct((8, D), jnp.float32),
    ]
    if s == 0:
        return pl.pallas_call(
            _edge1_body0,
            grid=(GS,),
            in_specs=[loc, glob, full(D, D), full(D, D), full(1, D)],
            out_specs=[loc, glob, full(8, D)],
            out_shape=out_shape,
        )(qk_s, edge_feat, we, ow, ob)
    return pl.pallas_call(
        _edge1_bodyN,
        grid=(GS,),
        in_specs=[loc, glob, anyspec, full(D, D), full(D, D), full(1, D)],
        out_specs=[loc, glob, full(8, D)],
        out_shape=out_shape,
        input_output_aliases={2: 1},
    )(qk_s, edge_feat, t_prev, we, ow, ob)


def _edge2_body(t_ref, acc_ref, w1_ref, b1_ref, w2_ref, b2_ref, g_ref, bb_ref,
                u_ref, acc2_ref):
    i = pl.program_id(0)
    a = acc_ref[...]
    mu = jnp.sum(a[:, 0, :], axis=0, keepdims=True) * (1.0 / E)
    var = jnp.sum(a[:, 1, :], axis=0, keepdims=True) * (1.0 / E) - mu * mu
    inv = g_ref[...] * jax.lax.rsqrt(var + 1e-5)
    e1 = (t_ref[...].astype(jnp.float32) - mu) * inv + bb_ref[...]
    hid = jnp.maximum(_bdot(e1, w1_ref[...]) + b1_ref[...], 0.0)
    u = e1 + _bdot(hid, w2_ref[...]) + b2_ref[...]
    u_ref[...] = u.astype(jnp.bfloat16)

    @pl.when(i == 0)
    def _():
        acc2_ref[...] = jnp.zeros_like(acc2_ref)

    acc2_ref[0:1, :] += jnp.sum(u, axis=0, keepdims=True)
    acc2_ref[1:2, :] += jnp.sum(u * u, axis=0, keepdims=True)


def _edge2_call(t, acc, w1, b1, w2, b2, g, bb):
    blk = lambda w: pl.BlockSpec((BE, w), lambda i: (i, 0))
    full = lambda r, c: pl.BlockSpec((r, c), lambda i: (0, 0))
    acc_spec = pl.BlockSpec((SL, 8, D), lambda i: (0, 0, 0))
    return pl.pallas_call(
        _edge2_body,
        grid=(GE,),
        in_specs=[blk(D), acc_spec, full(D, 2 * D), full(1, 2 * D),
                  full(2 * D, D), full(1, D), full(1, D), full(1, D)],
        out_specs=[blk(D), full(8, D)],
        out_shape=[
            jax.ShapeDtypeStruct((E, D), jnp.bfloat16),
            jax.ShapeDtypeStruct((8, D), jnp.float32),
        ],
    )(t, acc, w1, b1, w2, b2, g, bb)


def _edge3_body(u_ref, acc_ref, g_ref, bb_ref, e_ref):
    mu = acc_ref[0:1, :] * (1.0 / E)
    var = acc_ref[1:2, :] * (1.0 / E) - mu * mu
    inv = g_ref[...] * jax.lax.rsqrt(var + 1e-5)
    e_ref[...] = (u_ref[...].astype(jnp.float32) - mu) * inv + bb_ref[...]


def _edge3_call(u, acc, g, bb):
    blk = lambda w: pl.BlockSpec((BE, w), lambda i: (i, 0))
    full = lambda r, c: pl.BlockSpec((r, c), lambda i: (0, 0))
    return pl.pallas_call(
        _edge3_body,
        grid=(GE,),
        in_specs=[blk(D), full(8, D), full(1, D), full(1, D)],
        out_specs=blk(D),
        out_shape=jax.ShapeDtypeStruct((E, D), jnp.float32),
    )(u, acc, g, bb)


def _node_body(zpa_ref, vpa_ref, nf_ref, ow_ref, ob_ref,
               w1_ref, b1_ref, w2_ref, b2_ref,
               g1_ref, bb1_ref, g2_ref, bb2_ref, h_ref):
    z = zpa_ref[:N, :] + zpa_ref[NPAD:NPAD + N, :]
    v = vpa_ref[:N, :] + vpa_ref[NPAD:NPAD + N, :]
    h_attn = v / z + 1e-6
    h = nf_ref[...] + jnp.dot(h_attn, ow_ref[...],
                              preferred_element_type=jnp.float32) + ob_ref[...]
    mu = jnp.mean(h, axis=0, keepdims=True)
    var = jnp.mean((h - mu) * (h - mu), axis=0, keepdims=True)
    h = g1_ref[...] * (h - mu) * jax.lax.rsqrt(var + 1e-5) + bb1_ref[...]
    hid = jnp.maximum(
        jnp.dot(h, w1_ref[...], preferred_element_type=jnp.float32) + b1_ref[...],
        0.0,
    )
    h2 = h + jnp.dot(hid, w2_ref[...], preferred_element_type=jnp.float32) + b2_ref[...]
    mu2 = jnp.mean(h2, axis=0, keepdims=True)
    var2 = jnp.mean((h2 - mu2) * (h2 - mu2), axis=0, keepdims=True)
    h_ref[...] = g2_ref[...] * (h2 - mu2) * jax.lax.rsqrt(var2 + 1e-5) + bb2_ref[...]


def _node_call(zpa, vpa, nf, ow, ob, w1, b1, w2, b2, g1, bb1, g2, bb2):
    return pl.pallas_call(
        _node_body,
        out_shape=jax.ShapeDtypeStruct((N, D), jnp.float32),
    )(zpa, vpa, nf, ow, ob, w1, b1, w2, b2, g1, bb1, g2, bb2)


def kernel(node_feat, edge_feat, edge_index, W_Q, W_K, W_V, W_E,
           O_h_W, O_h_b, O_e_W, O_e_b,
           F_h_W1, F_h_b1, F_h_W2, F_h_b2,
           F_e_W1, F_e_b1, F_e_W2, F_e_b2,
           bn1_h_g, bn1_h_b, bn1_e_g, bn1_e_b,
           bn2_h_g, bn2_h_b, bn2_e_g, bn2_e_b):
    src = edge_index[0].astype(jnp.int32)
    dst = edge_index[1].astype(jnp.int32)
    r = lambda x: x.reshape(1, -1)

    q_tab, k_tab, v_tab = _qkv_call(node_feat, W_Q, W_K, W_V)

    dst_s = [dst[s * ES:(s + 1) * ES] for s in range(SL)]
    src_s = [src[s * ES:(s + 1) * ES] for s in range(SL)]
    qk_s = [_sc_gather(q_tab, k_tab, dst_s[s], src_s[s]) for s in range(SL)]

    t = None
    ef_s = []
    accs = []
    for s in range(SL):
        ef_i, t, acc_i = _edge1_call(s, qk_s[s], edge_feat, t,
                                     W_E, O_e_W, r(O_e_b))
        ef_s.append(ef_i)
        accs.append(acc_i)
    acc1 = jnp.stack(accs)

    zeros = jnp.zeros((CH, D), jnp.float32)
    zpa = _sc_scatter(tuple(ef_s), tuple(dst_s), zeros)
    vpa = _sc_scatter_mul(tuple(ef_s), v_tab, tuple(dst_s),
                          tuple(src_s), zeros)

    u, acc2 = _edge2_call(t, acc1, F_e_W1, r(F_e_b1), F_e_W2, r(F_e_b2),
                          r(bn1_e_g), r(bn1_e_b))
    e_out = _edge3_call(u, acc2, r(bn2_e_g), r(bn2_e_b))

    h_out = _node_call(zpa, vpa, node_feat, O_h_W, r(O_h_b),
                       F_h_W1, r(F_h_b1), F_h_W2, r(F_h_b2),
                       r(bn1_h_g), r(bn1_h_b), r(bn2_h_g), r(bn2_h_b))
    return (h_out, e_out)
